# Initial kernel scaffold; baseline (speedup 1.0000x reference)
import jax
import jax.numpy as jnp
from jax.experimental import pallas as pl


def kernel(x, weight, bias):
    raise NotImplementedError("write your pallas kernel here")


# speedup vs baseline: 1.3183x; 1.3183x over previous
import jax
import jax.numpy as jnp
from jax.experimental import pallas as pl
from jax.experimental.pallas import tpu as pltpu


def _pick_node_block(d, m1, m2, max_td=32):
    for td in range(1, max_td + 1):
        if (td * m1) % 128 == 0 and (td * m2) % 128 == 0:
            return td
    return d


def _make_body(gd, td, m1, m2, have_bias):
    km1 = td * m1
    km2 = td * m2

    def body(x_ref, w_ref, b_ref, o_ref):
        for g in range(gd):
            h = jnp.dot(
                x_ref[:, g * km1:(g + 1) * km1],
                w_ref[g],
                preferred_element_type=jnp.float32,
            )
            if have_bias:
                h = h + b_ref[:, g * km2:(g + 1) * km2].astype(jnp.float32)
            o_ref[:, g * km2:(g + 1) * km2] = h.astype(o_ref.dtype)

    def body_nobias(x_ref, w_ref, o_ref):
        body(x_ref, w_ref, None, o_ref)

    return body if have_bias else body_nobias


def kernel(x, weight, bias):
    n, d, m1 = x.shape
    d_w, m1_w, m2 = weight.shape
    assert d == d_w and m1 == m1_w

    td = _pick_node_block(d, m1, m2)
    gd = pl.cdiv(d, td)
    d_pad = gd * td

    tn = 512
    if n <= tn:
        tn = n
    gn = pl.cdiv(n, tn)
    n_pad = gn * tn

    if d_pad != d:
        x = jnp.pad(x, ((0, 0), (0, d_pad - d), (0, 0)))
        weight = jnp.pad(weight, ((0, d_pad - d), (0, 0), (0, 0)))
        if bias is not None:
            bias = jnp.pad(bias, ((0, d_pad - d), (0, 0)))
    if n_pad != n:
        x = jnp.pad(x, ((0, n_pad - n), (0, 0), (0, 0)))

    x2d = x.reshape(n_pad, d_pad * m1)

    eye = jnp.eye(td, dtype=weight.dtype)
    w_bd = jnp.einsum('gtio,ts->gtiso', weight.reshape(gd, td, m1, m2), eye)
    w_bd = w_bd.reshape(gd, td * m1, td * m2)

    x_spec = pl.BlockSpec((tn, d_pad * m1), lambda jn: (jn, 0))
    w_spec = pl.BlockSpec((gd, td * m1, td * m2), lambda jn: (0, 0, 0))
    o_spec = pl.BlockSpec((tn, d_pad * m2), lambda jn: (jn, 0))
    out_shape = jax.ShapeDtypeStruct((n_pad, d_pad * m2), x.dtype)

    itemsize = jnp.dtype(x.dtype).itemsize
    cost = pl.CostEstimate(
        flops=int(2 * n_pad * d_pad * td * m1 * m2),
        transcendentals=0,
        bytes_accessed=int((x2d.size + w_bd.size + n_pad * d_pad * m2
                            + (d_pad * m2 if bias is not None else 0)) * itemsize),
    )
    cparams = pltpu.CompilerParams(
        dimension_semantics=("parallel",),
        vmem_limit_bytes=100 * 1024 * 1024,
    )

    body = _make_body(gd, td, m1, m2, bias is not None)
    if bias is not None:
        b2d = bias.reshape(1, d_pad * m2)
        b_spec = pl.BlockSpec((1, d_pad * m2), lambda jn: (0, 0))
        out2d = pl.pallas_call(
            body,
            out_shape=out_shape,
            grid=(gn,),
            in_specs=[x_spec, w_spec, b_spec],
            out_specs=o_spec,
            compiler_params=cparams,
            cost_estimate=cost,
        )(x2d, w_bd, b2d)
    else:
        out2d = pl.pallas_call(
            body,
            out_shape=out_shape,
            grid=(gn,),
            in_specs=[x_spec, w_spec],
            out_specs=o_spec,
            compiler_params=cparams,
            cost_estimate=cost,
        )(x2d, w_bd)

    return out2d.reshape(n_pad, d_pad, m2)[:n, :d, :]
